# baseline (speedup 1.0000x reference)
import jax
import jax.numpy as jnp
from jax import lax
from jax.experimental import pallas as pl
from jax.experimental.pallas import tpu as pltpu

P = 196


def _body(in_ref, *rest):
    out_refs = rest[:P]
    s0, s1, sem0, sem1 = rest[P:]
    scratch = (s0, s1)
    sems = (sem0, sem1)
    a = in_ref[:, :, 0:P - 1]
    b = in_ref[:, :, 1:P]
    lane = lax.broadcasted_iota(jnp.int32, (4, 192, P - 1), 2)
    dmas = [None, None]
    for i in range(P):
        p = i % 2
        if dmas[p] is not None:
            dmas[p].wait()
        scratch[p][...] = jnp.where(lane < i, a, b)
        dma = pltpu.make_async_copy(scratch[p], out_refs[i], sems[p])
        dma.start()
        dmas[p] = dma
    for p in range(2):
        if dmas[p] is not None:
            dmas[p].wait()


_call = pl.pallas_call(
    _body,
    in_specs=[pl.BlockSpec(memory_space=pltpu.VMEM)],
    out_specs=tuple(pl.BlockSpec(memory_space=pl.ANY) for _ in range(P)),
    out_shape=tuple(jax.ShapeDtypeStruct((4, 192, P - 1), jnp.float32)
                    for _ in range(P)),
    scratch_shapes=[pltpu.VMEM((4, 192, P - 1), jnp.float32),
                    pltpu.VMEM((4, 192, P - 1), jnp.float32),
                    pltpu.SemaphoreType.DMA,
                    pltpu.SemaphoreType.DMA],
)


def kernel(patches):
    pt = jnp.transpose(patches, (1, 2, 0))
    outs = _call(pt)
    return tuple(jnp.transpose(o, (2, 0, 1)) for o in outs)


# speedup vs baseline: 1.9990x; 1.9990x over previous
import jax
import jax.numpy as jnp
from jax import lax
from jax.experimental import pallas as pl
from jax.experimental.pallas import tpu as pltpu

P = 196


NBUF = 6


def _body(in_ref, *rest):
    out_refs = rest[:P]
    scratch = rest[P:P + NBUF]
    sems = rest[P + NBUF:P + 2 * NBUF]
    a = in_ref[:, :, 0:P - 1]
    b = in_ref[:, :, 1:P]
    lane = lax.broadcasted_iota(jnp.int32, (4, 192, P - 1), 2)
    dmas = [None] * NBUF
    for i in range(P):
        p = i % NBUF
        if dmas[p] is not None:
            dmas[p].wait()
        scratch[p][...] = jnp.where(lane < i, a, b)
        dma = pltpu.make_async_copy(scratch[p], out_refs[i], sems[p])
        dma.start()
        dmas[p] = dma
    for p in range(NBUF):
        if dmas[p] is not None:
            dmas[p].wait()


_call = pl.pallas_call(
    _body,
    in_specs=[pl.BlockSpec(memory_space=pltpu.VMEM)],
    out_specs=tuple(pl.BlockSpec(memory_space=pl.ANY) for _ in range(P)),
    out_shape=tuple(jax.ShapeDtypeStruct((4, 192, P - 1), jnp.float32)
                    for _ in range(P)),
    scratch_shapes=([pltpu.VMEM((4, 192, P - 1), jnp.float32)] * NBUF
                    + [pltpu.SemaphoreType.DMA] * NBUF),
)


def kernel(patches):
    pt = jnp.transpose(patches, (1, 2, 0))
    outs = _call(pt)
    return tuple(jnp.transpose(o, (2, 0, 1)) for o in outs)
